# baseline (speedup 1.0000x reference)
import jax
import jax.numpy as jnp
import numpy as np
from jax.experimental import pallas as pl
from jax.experimental.pallas import tpu as pltpu

N = 10000
NCOL = 64
CH = 2048
NB = 5
RH = 1024
NRB = 10
LASTR = N - (NRB - 1) * RH
MAINW = 1792
TAILC = (NB - 1) * CH + MAINW
TAILW = N - TAILC
NPAD = NB * CH
SWEEP = NRB * NB
D_BLK = 400


def _schedule():
    rows, cols = [], []
    for j in range(NRB):
        dj = j // 2
        for t in range(NB):
            rows.append(j)
            cols.append((dj + 1 + t) % NB)
    for j in range(NRB):
        for k in range(NB):
            if 2 * k + 1 > j:
                rows.append(j)
                cols.append(k)
    rows.append(rows[-1])
    cols.append(cols[-1])
    return np.asarray(rows, np.int32), np.asarray(cols, np.int32)


_SCHED_R, _SCHED_C = _schedule()
STEPS = len(_SCHED_R) - 1


def _y_kernel(feat_ref, w1_ref, w2_ref, y_ref, ytail_ref):
    w12 = jnp.dot(w1_ref[...], w2_ref[...], preferred_element_type=jnp.float32)
    yfull = jnp.dot(feat_ref[...], w12, preferred_element_type=jnp.float32)
    y_ref[pl.ds(0, TAILC), :] = yfull[:TAILC, :]
    y_ref[pl.ds(TAILC, NPAD - TAILC), :] = jnp.zeros(
        (NPAD - TAILC, NCOL), jnp.float32)
    ytail_ref[...] = yfull[TAILC:, :]


def _mega_kernel(rs_ref, cs_ref, y_ref, ytail_ref, adj_ref,
                 x_out_ref, abuf, tbuf, x3buf, x4buf, x3tail, sem, tsem):
    s = pl.program_id(0)
    lastc = NB - 1
    lastr = NRB - 1

    def issue(t, slot, start):
        r = rs_ref[t]
        c = cs_ref[t]
        for hr, rcond in ((RH, r < lastr), (LASTR, r == lastr)):
            @pl.when(jnp.logical_and(rcond, c < lastc))
            def _():
                cp = pltpu.make_async_copy(
                    adj_ref.at[pl.ds(r * RH, hr), pl.ds(c * CH, CH)],
                    abuf.at[slot, pl.ds(0, hr), :],
                    sem.at[slot])
                cp.start() if start else cp.wait()

            @pl.when(jnp.logical_and(rcond, c == lastc))
            def _():
                cp = pltpu.make_async_copy(
                    adj_ref.at[pl.ds(r * RH, hr), pl.ds(lastc * CH, MAINW)],
                    abuf.at[slot, pl.ds(0, hr), pl.ds(0, MAINW)],
                    sem.at[slot])
                cp.start() if start else cp.wait()

    @pl.when(s == 0)
    def _():
        issue(0, 0, True)
        tp = pltpu.make_async_copy(
            adj_ref.at[pl.ds(0, N), pl.ds(TAILC, TAILW)],
            tbuf.at[pl.ds(0, N), :], tsem)
        tp.start()
        tp.wait()
        for b in range(NB):
            x3buf[pl.ds(b * CH, CH), :] = jnp.zeros((CH, NCOL), jnp.float32)
            x4buf[pl.ds(b * CH, CH), :] = jnp.zeros((CH, NCOL), jnp.float32)

    slot = jax.lax.rem(s, 2)
    nslot = jax.lax.rem(s + 1, 2)

    @pl.when(s + 1 < STEPS)
    def _():
        issue(s + 1, nslot, True)

    issue(s, slot, False)

    r = rs_ref[s]
    c = cs_ref[s]
    roff = pl.multiple_of(r * RH, 8)
    coff = pl.multiple_of(c * CH, 8)
    blk = abuf[slot]
    is_tail_col = c == lastc

    @pl.when(s < SWEEP)
    def _():
        x3buf[pl.ds(roff, RH), :] = x3buf[pl.ds(roff, RH), :] + jnp.dot(
            blk, y_ref[pl.ds(coff, CH), :],
            preferred_element_type=jnp.float32)

        @pl.when(is_tail_col)
        def _():
            x3buf[pl.ds(roff, RH), :] = x3buf[pl.ds(roff, RH), :] + jnp.dot(
                tbuf[pl.ds(roff, RH), :], ytail_ref[...],
                preferred_element_type=jnp.float32)

    @pl.when(s == SWEEP - 1)
    def _():
        x3tail[...] = x3buf[pl.ds(TAILC, TAILW), :]
        x3buf[pl.ds(TAILC, NPAD - TAILC), :] = jnp.zeros(
            (NPAD - TAILC, NCOL), jnp.float32)

    @pl.when(jnp.logical_or(s >= SWEEP, 2 * c + 1 <= r))
    def _():
        x4buf[pl.ds(roff, RH), :] = x4buf[pl.ds(roff, RH), :] + jnp.dot(
            blk, x3buf[pl.ds(coff, CH), :],
            preferred_element_type=jnp.float32)

        @pl.when(is_tail_col)
        def _():
            x4buf[pl.ds(roff, RH), :] = x4buf[pl.ds(roff, RH), :] + jnp.dot(
                tbuf[pl.ds(roff, RH), :], x3tail[...],
                preferred_element_type=jnp.float32)

    @pl.when(s == STEPS - 1)
    def _():
        for b in range(NB - 1):
            x_out_ref[pl.ds(b * CH, CH), :] = x4buf[pl.ds(b * CH, CH), :]
        x_out_ref[pl.ds((NB - 1) * CH, N - (NB - 1) * CH), :] = x4buf[
            pl.ds((NB - 1) * CH, N - (NB - 1) * CH), :]


def _decoder_kernel(xr_ref, xc_ref, out_ref):
    z = jax.lax.dot_general(xr_ref[...], xc_ref[...],
                            (((1,), (1,)), ((), ())),
                            preferred_element_type=jnp.float32)
    out_ref[...] = 0.5 * (jnp.tanh(0.5 * z) + 1.0)


def kernel(feat, adj, W1, W2):
    y_pad, y_tail = pl.pallas_call(
        _y_kernel,
        out_shape=[
            jax.ShapeDtypeStruct((NPAD, NCOL), jnp.float32),
            jax.ShapeDtypeStruct((TAILW, NCOL), jnp.float32),
        ],
    )(feat, W1, W2)

    x = pl.pallas_call(
        _mega_kernel,
        grid_spec=pltpu.PrefetchScalarGridSpec(
            num_scalar_prefetch=2,
            grid=(STEPS,),
            in_specs=[
                pl.BlockSpec((NPAD, NCOL), lambda s, rs, cs: (0, 0)),
                pl.BlockSpec((TAILW, NCOL), lambda s, rs, cs: (0, 0)),
                pl.BlockSpec(memory_space=pl.ANY),
            ],
            out_specs=pl.BlockSpec((N, NCOL), lambda s, rs, cs: (0, 0)),
            scratch_shapes=[
                pltpu.VMEM((2, RH, CH), jnp.float32),
                pltpu.VMEM((NPAD, TAILW), jnp.float32),
                pltpu.VMEM((NPAD, NCOL), jnp.float32),
                pltpu.VMEM((NPAD, NCOL), jnp.float32),
                pltpu.VMEM((TAILW, NCOL), jnp.float32),
                pltpu.SemaphoreType.DMA((2,)),
                pltpu.SemaphoreType.DMA,
            ],
        ),
        out_shape=jax.ShapeDtypeStruct((N, NCOL), jnp.float32),
    )(jnp.asarray(_SCHED_R), jnp.asarray(_SCHED_C), y_pad, y_tail, adj)

    a_rec = pl.pallas_call(
        _decoder_kernel,
        grid=(N // D_BLK,),
        in_specs=[
            pl.BlockSpec((D_BLK, NCOL), lambda i: (i, 0)),
            pl.BlockSpec((N, NCOL), lambda i: (0, 0)),
        ],
        out_specs=pl.BlockSpec((D_BLK, N), lambda i: (i, 0)),
        out_shape=jax.ShapeDtypeStruct((N, N), jnp.float32),
    )(x, x)

    return (x, a_rec)


# speedup vs baseline: 1.0104x; 1.0104x over previous
import jax
import jax.numpy as jnp
import numpy as np
from jax.experimental import pallas as pl
from jax.experimental.pallas import tpu as pltpu

N = 10000
NCOL = 64
CH = 2048
NB = 5
RH = 1024
NRB = 10
LASTR = N - (NRB - 1) * RH
MAINW = 1792
TAILC = (NB - 1) * CH + MAINW
TAILW = N - TAILC
NPAD = NB * CH
SWEEP = NRB * NB
D_BLK = 400


def _schedule():
    rows, cols = [], []
    for j in range(NRB):
        dj = j // 2
        for t in range(NB):
            rows.append(j)
            cols.append((dj + 1 + t) % NB)
    for j in range(NRB):
        for k in range(NB):
            if 2 * k + 1 > j:
                rows.append(j)
                cols.append(k)
    rows.append(rows[-1])
    cols.append(cols[-1])
    return np.asarray(rows, np.int32), np.asarray(cols, np.int32)


_SCHED_R, _SCHED_C = _schedule()
STEPS = len(_SCHED_R) - 1


def _y_kernel(feat_ref, w1_ref, w2_ref, y_ref, ytail_ref):
    w12 = jnp.dot(w1_ref[...], w2_ref[...], preferred_element_type=jnp.float32)
    yfull = jnp.dot(feat_ref[...], w12, preferred_element_type=jnp.float32)
    y_ref[pl.ds(0, TAILC), :] = yfull[:TAILC, :]
    y_ref[pl.ds(TAILC, NPAD - TAILC), :] = jnp.zeros(
        (NPAD - TAILC, NCOL), jnp.float32)
    ytail_ref[...] = yfull[TAILC:, :]


def _mega_kernel(rs_ref, cs_ref, y_ref, ytail_ref, adj_ref,
                 x_out_ref, abuf, tbuf, x3buf, x4buf, x3tail, sem, tsem):
    s = pl.program_id(0)
    lastc = NB - 1
    lastr = NRB - 1

    def issue(t, slot, start):
        r = rs_ref[t]
        c = cs_ref[t]
        for hr, rcond in ((RH, r < lastr), (LASTR, r == lastr)):
            @pl.when(jnp.logical_and(rcond, c < lastc))
            def _():
                cp = pltpu.make_async_copy(
                    adj_ref.at[pl.ds(r * RH, hr), pl.ds(c * CH, CH)],
                    abuf.at[slot, pl.ds(0, hr), :],
                    sem.at[slot])
                cp.start() if start else cp.wait()

            @pl.when(jnp.logical_and(rcond, c == lastc))
            def _():
                cp = pltpu.make_async_copy(
                    adj_ref.at[pl.ds(r * RH, hr), pl.ds(lastc * CH, MAINW)],
                    abuf.at[slot, pl.ds(0, hr), pl.ds(0, MAINW)],
                    sem.at[slot])
                cp.start() if start else cp.wait()

    @pl.when(s == 0)
    def _():
        issue(0, 0, True)
        tp = pltpu.make_async_copy(
            adj_ref.at[pl.ds(0, N), pl.ds(TAILC, TAILW)],
            tbuf.at[pl.ds(0, N), :], tsem)
        tp.start()
        tp.wait()
        for b in range(NB):
            x3buf[pl.ds(b * CH, CH), :] = jnp.zeros((CH, NCOL), jnp.float32)
            x4buf[pl.ds(b * CH, CH), :] = jnp.zeros((CH, NCOL), jnp.float32)

    slot = jax.lax.rem(s, 2)
    nslot = jax.lax.rem(s + 1, 2)

    @pl.when(s + 1 < STEPS)
    def _():
        issue(s + 1, nslot, True)

    issue(s, slot, False)

    r = rs_ref[s]
    c = cs_ref[s]
    roff = pl.multiple_of(r * RH, 8)
    coff = pl.multiple_of(c * CH, 8)
    blk = abuf[slot]
    is_tail_col = c == lastc

    @pl.when(s < SWEEP)
    def _():
        x3buf[pl.ds(roff, RH), :] = x3buf[pl.ds(roff, RH), :] + jnp.dot(
            blk, y_ref[pl.ds(coff, CH), :],
            preferred_element_type=jnp.float32,
            precision=jax.lax.Precision.DEFAULT)

        @pl.when(is_tail_col)
        def _():
            x3buf[pl.ds(roff, RH), :] = x3buf[pl.ds(roff, RH), :] + jnp.dot(
                tbuf[pl.ds(roff, RH), :], ytail_ref[...],
                preferred_element_type=jnp.float32)

    @pl.when(s == SWEEP - 1)
    def _():
        x3tail[...] = x3buf[pl.ds(TAILC, TAILW), :]
        x3buf[pl.ds(TAILC, NPAD - TAILC), :] = jnp.zeros(
            (NPAD - TAILC, NCOL), jnp.float32)

    @pl.when(jnp.logical_or(s >= SWEEP, 2 * c + 1 <= r))
    def _():
        x4buf[pl.ds(roff, RH), :] = x4buf[pl.ds(roff, RH), :] + jnp.dot(
            blk, x3buf[pl.ds(coff, CH), :],
            preferred_element_type=jnp.float32,
            precision=jax.lax.Precision.DEFAULT)

        @pl.when(is_tail_col)
        def _():
            x4buf[pl.ds(roff, RH), :] = x4buf[pl.ds(roff, RH), :] + jnp.dot(
                tbuf[pl.ds(roff, RH), :], x3tail[...],
                preferred_element_type=jnp.float32)

    @pl.when(s == STEPS - 1)
    def _():
        for b in range(NB - 1):
            x_out_ref[pl.ds(b * CH, CH), :] = x4buf[pl.ds(b * CH, CH), :]
        x_out_ref[pl.ds((NB - 1) * CH, N - (NB - 1) * CH), :] = x4buf[
            pl.ds((NB - 1) * CH, N - (NB - 1) * CH), :]


def _decoder_kernel(xr_ref, xc_ref, out_ref):
    z = jax.lax.dot_general(xr_ref[...], xc_ref[...],
                            (((1,), (1,)), ((), ())),
                            preferred_element_type=jnp.float32)
    out_ref[...] = 0.5 * (jnp.tanh(0.5 * z) + 1.0)


def kernel(feat, adj, W1, W2):
    y_pad, y_tail = pl.pallas_call(
        _y_kernel,
        out_shape=[
            jax.ShapeDtypeStruct((NPAD, NCOL), jnp.float32),
            jax.ShapeDtypeStruct((TAILW, NCOL), jnp.float32),
        ],
    )(feat, W1, W2)

    x = pl.pallas_call(
        _mega_kernel,
        grid_spec=pltpu.PrefetchScalarGridSpec(
            num_scalar_prefetch=2,
            grid=(STEPS,),
            in_specs=[
                pl.BlockSpec((NPAD, NCOL), lambda s, rs, cs: (0, 0)),
                pl.BlockSpec((TAILW, NCOL), lambda s, rs, cs: (0, 0)),
                pl.BlockSpec(memory_space=pl.ANY),
            ],
            out_specs=pl.BlockSpec((N, NCOL), lambda s, rs, cs: (0, 0)),
            scratch_shapes=[
                pltpu.VMEM((2, RH, CH), jnp.float32),
                pltpu.VMEM((NPAD, TAILW), jnp.float32),
                pltpu.VMEM((NPAD, NCOL), jnp.float32),
                pltpu.VMEM((NPAD, NCOL), jnp.float32),
                pltpu.VMEM((TAILW, NCOL), jnp.float32),
                pltpu.SemaphoreType.DMA((2,)),
                pltpu.SemaphoreType.DMA,
            ],
        ),
        out_shape=jax.ShapeDtypeStruct((N, NCOL), jnp.float32),
    )(jnp.asarray(_SCHED_R), jnp.asarray(_SCHED_C), y_pad, y_tail, adj)

    a_rec = pl.pallas_call(
        _decoder_kernel,
        grid=(N // D_BLK,),
        in_specs=[
            pl.BlockSpec((D_BLK, NCOL), lambda i: (i, 0)),
            pl.BlockSpec((N, NCOL), lambda i: (0, 0)),
        ],
        out_specs=pl.BlockSpec((D_BLK, N), lambda i: (i, 0)),
        out_shape=jax.ShapeDtypeStruct((N, N), jnp.float32),
    )(x, x)

    return (x, a_rec)


# speedup vs baseline: 1.0460x; 1.0353x over previous
import jax
import jax.numpy as jnp
import numpy as np
from jax.experimental import pallas as pl
from jax.experimental.pallas import tpu as pltpu

N = 10000
NCOL = 64
CH = 2048
NB = 5
RH = 1024
NRB = 10
LASTR = N - (NRB - 1) * RH
MAINW = 1792
TAILC = (NB - 1) * CH + MAINW
TAILW = N - TAILC
NPAD = NB * CH
SWEEP = NRB * NB
D_BLK = 400


def _schedule():
    rows, cols = [], []
    for j in range(NRB):
        dj = j // 2
        for t in range(NB):
            rows.append(j)
            cols.append((dj + 1 + t) % NB)
    for j in range(NRB):
        for k in range(NB):
            if 2 * k + 1 > j:
                rows.append(j)
                cols.append(k)
    rows.append(rows[-1])
    cols.append(cols[-1])
    return np.asarray(rows, np.int32), np.asarray(cols, np.int32)


_SCHED_R, _SCHED_C = _schedule()
STEPS = len(_SCHED_R) - 1


def _y_kernel(feat_ref, w1_ref, w2_ref, y_ref, ytail_ref):
    w12 = jnp.dot(w1_ref[...], w2_ref[...], preferred_element_type=jnp.float32)
    yfull = jnp.dot(feat_ref[...], w12, preferred_element_type=jnp.float32)
    y_ref[pl.ds(0, TAILC), :] = yfull[:TAILC, :]
    y_ref[pl.ds(TAILC, NPAD - TAILC), :] = jnp.zeros(
        (NPAD - TAILC, NCOL), jnp.float32)
    ytail_ref[...] = yfull[TAILC:, :]


def _mega_kernel(rs_ref, cs_ref, y_ref, ytail_ref, adj_ref,
                 x_out_ref, abuf, tbuf, x3buf, x4buf, x3tail, sem, tsem):
    s = pl.program_id(0)
    lastc = NB - 1
    lastr = NRB - 1

    def issue(t, slot, start):
        r = rs_ref[t]
        c = cs_ref[t]
        for hr, rcond in ((RH, r < lastr), (LASTR, r == lastr)):
            @pl.when(jnp.logical_and(rcond, c < lastc))
            def _():
                cp = pltpu.make_async_copy(
                    adj_ref.at[pl.ds(r * RH, hr), pl.ds(c * CH, CH)],
                    abuf.at[slot, pl.ds(0, hr), :],
                    sem.at[slot])
                cp.start() if start else cp.wait()

            @pl.when(jnp.logical_and(rcond, c == lastc))
            def _():
                cp = pltpu.make_async_copy(
                    adj_ref.at[pl.ds(r * RH, hr), pl.ds(lastc * CH, MAINW)],
                    abuf.at[slot, pl.ds(0, hr), pl.ds(0, MAINW)],
                    sem.at[slot])
                cp.start() if start else cp.wait()

    @pl.when(s == 0)
    def _():
        issue(0, 0, True)
        tp = pltpu.make_async_copy(
            adj_ref.at[pl.ds(0, N), pl.ds(TAILC, TAILW)],
            tbuf.at[pl.ds(0, N), :], tsem)
        tp.start()
        tp.wait()
        for b in range(NB):
            x3buf[pl.ds(b * CH, CH), :] = jnp.zeros((CH, NCOL), jnp.float32)
            x4buf[pl.ds(b * CH, CH), :] = jnp.zeros((CH, NCOL), jnp.float32)

    slot = jax.lax.rem(s, 2)
    nslot = jax.lax.rem(s + 1, 2)

    @pl.when(s + 1 < STEPS)
    def _():
        issue(s + 1, nslot, True)

    issue(s, slot, False)

    r = rs_ref[s]
    c = cs_ref[s]
    roff = pl.multiple_of(r * RH, 8)
    coff = pl.multiple_of(c * CH, 8)
    is_tail_col = c == lastc
    do_x4 = jnp.logical_or(s >= SWEEP, 2 * c + 1 <= r)

    for sl in (0, 1):
        @pl.when(jnp.logical_and(slot == sl, s < SWEEP))
        def _(sl=sl):
            x3buf[pl.ds(roff, RH), :] = x3buf[pl.ds(roff, RH), :] + jnp.dot(
                abuf[sl], y_ref[pl.ds(coff, CH), :],
                preferred_element_type=jnp.float32,
                precision=jax.lax.Precision.DEFAULT)

    @pl.when(jnp.logical_and(is_tail_col, s < SWEEP))
    def _():
        x3buf[pl.ds(roff, RH), :] = x3buf[pl.ds(roff, RH), :] + jnp.dot(
            tbuf[pl.ds(roff, RH), :], ytail_ref[...],
            preferred_element_type=jnp.float32)

    @pl.when(s == SWEEP - 1)
    def _():
        x3tail[...] = x3buf[pl.ds(TAILC, TAILW), :]
        x3buf[pl.ds(TAILC, NPAD - TAILC), :] = jnp.zeros(
            (NPAD - TAILC, NCOL), jnp.float32)

    for sl in (0, 1):
        @pl.when(jnp.logical_and(slot == sl, do_x4))
        def _(sl=sl):
            x4buf[pl.ds(roff, RH), :] = x4buf[pl.ds(roff, RH), :] + jnp.dot(
                abuf[sl], x3buf[pl.ds(coff, CH), :],
                preferred_element_type=jnp.float32,
                precision=jax.lax.Precision.DEFAULT)

    @pl.when(jnp.logical_and(is_tail_col, do_x4))
    def _():
        x4buf[pl.ds(roff, RH), :] = x4buf[pl.ds(roff, RH), :] + jnp.dot(
            tbuf[pl.ds(roff, RH), :], x3tail[...],
            preferred_element_type=jnp.float32)

    @pl.when(s == STEPS - 1)
    def _():
        for b in range(NB - 1):
            x_out_ref[pl.ds(b * CH, CH), :] = x4buf[pl.ds(b * CH, CH), :]
        x_out_ref[pl.ds((NB - 1) * CH, N - (NB - 1) * CH), :] = x4buf[
            pl.ds((NB - 1) * CH, N - (NB - 1) * CH), :]


def _decoder_kernel(xr_ref, xc_ref, out_ref):
    z = jax.lax.dot_general(xr_ref[...], xc_ref[...],
                            (((1,), (1,)), ((), ())),
                            preferred_element_type=jnp.float32)
    out_ref[...] = 0.5 * (jnp.tanh(0.5 * z) + 1.0)


def kernel(feat, adj, W1, W2):
    y_pad, y_tail = pl.pallas_call(
        _y_kernel,
        out_shape=[
            jax.ShapeDtypeStruct((NPAD, NCOL), jnp.float32),
            jax.ShapeDtypeStruct((TAILW, NCOL), jnp.float32),
        ],
    )(feat, W1, W2)

    x = pl.pallas_call(
        _mega_kernel,
        grid_spec=pltpu.PrefetchScalarGridSpec(
            num_scalar_prefetch=2,
            grid=(STEPS,),
            in_specs=[
                pl.BlockSpec((NPAD, NCOL), lambda s, rs, cs: (0, 0)),
                pl.BlockSpec((TAILW, NCOL), lambda s, rs, cs: (0, 0)),
                pl.BlockSpec(memory_space=pl.ANY),
            ],
            out_specs=pl.BlockSpec((N, NCOL), lambda s, rs, cs: (0, 0)),
            scratch_shapes=[
                pltpu.VMEM((2, RH, CH), jnp.float32),
                pltpu.VMEM((NPAD, TAILW), jnp.float32),
                pltpu.VMEM((NPAD, NCOL), jnp.float32),
                pltpu.VMEM((NPAD, NCOL), jnp.float32),
                pltpu.VMEM((TAILW, NCOL), jnp.float32),
                pltpu.SemaphoreType.DMA((2,)),
                pltpu.SemaphoreType.DMA,
            ],
        ),
        out_shape=jax.ShapeDtypeStruct((N, NCOL), jnp.float32),
    )(jnp.asarray(_SCHED_R), jnp.asarray(_SCHED_C), y_pad, y_tail, adj)

    a_rec = pl.pallas_call(
        _decoder_kernel,
        grid=(N // D_BLK,),
        in_specs=[
            pl.BlockSpec((D_BLK, NCOL), lambda i: (i, 0)),
            pl.BlockSpec((N, NCOL), lambda i: (0, 0)),
        ],
        out_specs=pl.BlockSpec((D_BLK, N), lambda i: (i, 0)),
        out_shape=jax.ShapeDtypeStruct((N, N), jnp.float32),
    )(x, x)

    return (x, a_rec)


# speedup vs baseline: 1.1212x; 1.0718x over previous
import jax
import jax.numpy as jnp
import numpy as np
from jax.experimental import pallas as pl
from jax.experimental.pallas import tpu as pltpu

N = 10000
NCOL = 64
CH = 2048
NB = 5
RH = 2048
NRB = 5
LASTR = N - (NRB - 1) * RH
MAINW = 1792
TAILC = (NB - 1) * CH + MAINW
TAILW = N - TAILC
NPAD = NB * CH
SWEEP = NRB * NB
D_BLK = 400


def _schedule():
    rows, cols = [], []
    for j in range(NRB):
        for t in range(NB):
            rows.append(j)
            cols.append((j + 1 + t) % NB)
    for j in range(NRB):
        for k in range(NB):
            if k > j:
                rows.append(j)
                cols.append(k)
    rows.append(rows[-1])
    cols.append(cols[-1])
    return np.asarray(rows, np.int32), np.asarray(cols, np.int32)


_SCHED_R, _SCHED_C = _schedule()
STEPS = len(_SCHED_R) - 1


def _y_kernel(feat_ref, w1_ref, w2_ref, y_ref, ytail_ref):
    w12 = jnp.dot(w1_ref[...], w2_ref[...], preferred_element_type=jnp.float32)
    yfull = jnp.dot(feat_ref[...], w12, preferred_element_type=jnp.float32)
    y_ref[pl.ds(0, TAILC), :] = yfull[:TAILC, :]
    y_ref[pl.ds(TAILC, NPAD - TAILC), :] = jnp.zeros(
        (NPAD - TAILC, NCOL), jnp.float32)
    ytail_ref[...] = yfull[TAILC:, :]


def _mega_kernel(rs_ref, cs_ref, y_ref, ytail_ref, adj_ref,
                 x_out_ref, abuf, tbuf, x3buf, x4buf, x3tail, sem, tsem):
    s = pl.program_id(0)
    lastc = NB - 1
    lastr = NRB - 1

    def issue(t, slot, start):
        r = rs_ref[t]
        c = cs_ref[t]
        for hr, rcond in ((RH, r < lastr), (LASTR, r == lastr)):
            @pl.when(jnp.logical_and(rcond, c < lastc))
            def _():
                cp = pltpu.make_async_copy(
                    adj_ref.at[pl.ds(r * RH, hr), pl.ds(c * CH, CH)],
                    abuf.at[slot, pl.ds(0, hr), :],
                    sem.at[slot])
                cp.start() if start else cp.wait()

            @pl.when(jnp.logical_and(rcond, c == lastc))
            def _():
                cp = pltpu.make_async_copy(
                    adj_ref.at[pl.ds(r * RH, hr), pl.ds(lastc * CH, MAINW)],
                    abuf.at[slot, pl.ds(0, hr), pl.ds(0, MAINW)],
                    sem.at[slot])
                cp.start() if start else cp.wait()

    @pl.when(s == 0)
    def _():
        issue(0, 0, True)
        tp = pltpu.make_async_copy(
            adj_ref.at[pl.ds(0, N), pl.ds(TAILC, TAILW)],
            tbuf.at[pl.ds(0, N), :], tsem)
        tp.start()
        tp.wait()
        for b in range(NB):
            x3buf[pl.ds(b * CH, CH), :] = jnp.zeros((CH, NCOL), jnp.float32)
            x4buf[pl.ds(b * CH, CH), :] = jnp.zeros((CH, NCOL), jnp.float32)

    slot = jax.lax.rem(s, 2)
    nslot = jax.lax.rem(s + 1, 2)

    @pl.when(s + 1 < STEPS)
    def _():
        issue(s + 1, nslot, True)

    issue(s, slot, False)

    r = rs_ref[s]
    c = cs_ref[s]
    roff = pl.multiple_of(r * RH, 8)
    coff = pl.multiple_of(c * CH, 8)
    is_tail_col = c == lastc
    do_x4 = jnp.logical_or(s >= SWEEP, c <= r)

    for sl in (0, 1):
        @pl.when(jnp.logical_and(slot == sl, s < SWEEP))
        def _(sl=sl):
            x3buf[pl.ds(roff, RH), :] = x3buf[pl.ds(roff, RH), :] + jnp.dot(
                abuf[sl], y_ref[pl.ds(coff, CH), :],
                preferred_element_type=jnp.float32,
                precision=jax.lax.Precision.DEFAULT)

    @pl.when(jnp.logical_and(is_tail_col, s < SWEEP))
    def _():
        x3buf[pl.ds(roff, RH), :] = x3buf[pl.ds(roff, RH), :] + jnp.dot(
            tbuf[pl.ds(roff, RH), :], ytail_ref[...],
            preferred_element_type=jnp.float32)

    @pl.when(s == SWEEP - 1)
    def _():
        x3tail[...] = x3buf[pl.ds(TAILC, TAILW), :]
        x3buf[pl.ds(TAILC, NPAD - TAILC), :] = jnp.zeros(
            (NPAD - TAILC, NCOL), jnp.float32)

    for sl in (0, 1):
        @pl.when(jnp.logical_and(slot == sl, do_x4))
        def _(sl=sl):
            x4buf[pl.ds(roff, RH), :] = x4buf[pl.ds(roff, RH), :] + jnp.dot(
                abuf[sl], x3buf[pl.ds(coff, CH), :],
                preferred_element_type=jnp.float32,
                precision=jax.lax.Precision.DEFAULT)

    @pl.when(jnp.logical_and(is_tail_col, do_x4))
    def _():
        x4buf[pl.ds(roff, RH), :] = x4buf[pl.ds(roff, RH), :] + jnp.dot(
            tbuf[pl.ds(roff, RH), :], x3tail[...],
            preferred_element_type=jnp.float32)

    @pl.when(s == STEPS - 1)
    def _():
        for b in range(NB - 1):
            x_out_ref[pl.ds(b * CH, CH), :] = x4buf[pl.ds(b * CH, CH), :]
        x_out_ref[pl.ds((NB - 1) * CH, N - (NB - 1) * CH), :] = x4buf[
            pl.ds((NB - 1) * CH, N - (NB - 1) * CH), :]


def _decoder_kernel(xr_ref, xc_ref, out_ref):
    z = jax.lax.dot_general(xr_ref[...], xc_ref[...],
                            (((1,), (1,)), ((), ())),
                            preferred_element_type=jnp.float32)
    out_ref[...] = 0.5 * (jnp.tanh(0.5 * z) + 1.0)


def kernel(feat, adj, W1, W2):
    y_pad, y_tail = pl.pallas_call(
        _y_kernel,
        out_shape=[
            jax.ShapeDtypeStruct((NPAD, NCOL), jnp.float32),
            jax.ShapeDtypeStruct((TAILW, NCOL), jnp.float32),
        ],
    )(feat, W1, W2)

    x = pl.pallas_call(
        _mega_kernel,
        grid_spec=pltpu.PrefetchScalarGridSpec(
            num_scalar_prefetch=2,
            grid=(STEPS,),
            in_specs=[
                pl.BlockSpec((NPAD, NCOL), lambda s, rs, cs: (0, 0)),
                pl.BlockSpec((TAILW, NCOL), lambda s, rs, cs: (0, 0)),
                pl.BlockSpec(memory_space=pl.ANY),
            ],
            out_specs=pl.BlockSpec((N, NCOL), lambda s, rs, cs: (0, 0)),
            scratch_shapes=[
                pltpu.VMEM((2, RH, CH), jnp.float32),
                pltpu.VMEM((NPAD, TAILW), jnp.float32),
                pltpu.VMEM((NPAD, NCOL), jnp.float32),
                pltpu.VMEM((NPAD, NCOL), jnp.float32),
                pltpu.VMEM((TAILW, NCOL), jnp.float32),
                pltpu.SemaphoreType.DMA((2,)),
                pltpu.SemaphoreType.DMA,
            ],
        ),
        out_shape=jax.ShapeDtypeStruct((N, NCOL), jnp.float32),
    )(jnp.asarray(_SCHED_R), jnp.asarray(_SCHED_C), y_pad, y_tail, adj)

    a_rec = pl.pallas_call(
        _decoder_kernel,
        grid=(N // D_BLK,),
        in_specs=[
            pl.BlockSpec((D_BLK, NCOL), lambda i: (i, 0)),
            pl.BlockSpec((N, NCOL), lambda i: (0, 0)),
        ],
        out_specs=pl.BlockSpec((D_BLK, N), lambda i: (i, 0)),
        out_shape=jax.ShapeDtypeStruct((N, N), jnp.float32),
    )(x, x)

    return (x, a_rec)


# speedup vs baseline: 1.1381x; 1.0150x over previous
import jax
import jax.numpy as jnp
import numpy as np
from jax.experimental import pallas as pl
from jax.experimental.pallas import tpu as pltpu

N = 10000
NCOL = 64
CH = 2048
NB = 5
RH = 2048
NRB = 5
LASTR = N - (NRB - 1) * RH
MAINW = 1792
TAILC = (NB - 1) * CH + MAINW
TAILW = N - TAILC
NPAD = NB * CH
SWEEP = NRB * NB
D_BLK = 400


def _schedule():
    rows, cols = [], []
    for j in range(NRB):
        for t in range(NB):
            rows.append(j)
            cols.append((j + 1 + t) % NB)
    for j in range(NRB):
        for k in range(NB):
            if k > j:
                rows.append(j)
                cols.append(k)
    rows.append(rows[-1])
    cols.append(cols[-1])
    return np.asarray(rows, np.int32), np.asarray(cols, np.int32)


_SCHED_R, _SCHED_C = _schedule()
STEPS = len(_SCHED_R) - 1


def _mega_kernel(rs_ref, cs_ref, feat_ref, w1_ref, w2_ref, adj_ref,
                 x_out_ref, abuf, tbuf, x3buf, x4buf, x3tail, y_ref,
                 ytail_ref, sem, tsem):
    s = pl.program_id(0)
    lastc = NB - 1
    lastr = NRB - 1

    def issue(t, slot, start):
        r = rs_ref[t]
        c = cs_ref[t]
        for hr, rcond in ((RH, r < lastr), (LASTR, r == lastr)):
            @pl.when(jnp.logical_and(rcond, c < lastc))
            def _():
                cp = pltpu.make_async_copy(
                    adj_ref.at[pl.ds(r * RH, hr), pl.ds(c * CH, CH)],
                    abuf.at[slot, pl.ds(0, hr), :],
                    sem.at[slot])
                cp.start() if start else cp.wait()

            @pl.when(jnp.logical_and(rcond, c == lastc))
            def _():
                cp = pltpu.make_async_copy(
                    adj_ref.at[pl.ds(r * RH, hr), pl.ds(lastc * CH, MAINW)],
                    abuf.at[slot, pl.ds(0, hr), pl.ds(0, MAINW)],
                    sem.at[slot])
                cp.start() if start else cp.wait()

    @pl.when(s == 0)
    def _():
        issue(0, 0, True)
        tp = pltpu.make_async_copy(
            adj_ref.at[pl.ds(0, N), pl.ds(TAILC, TAILW)],
            tbuf.at[pl.ds(0, N), :], tsem)
        tp.start()
        w12 = jnp.dot(w1_ref[...], w2_ref[...],
                      preferred_element_type=jnp.float32)
        yfull = jnp.dot(feat_ref[...], w12, preferred_element_type=jnp.float32)
        y_ref[pl.ds(0, TAILC), :] = yfull[:TAILC, :]
        y_ref[pl.ds(TAILC, NPAD - TAILC), :] = jnp.zeros(
            (NPAD - TAILC, NCOL), jnp.float32)
        ytail_ref[...] = yfull[TAILC:, :]
        tp.wait()
        for b in range(NB):
            x3buf[pl.ds(b * CH, CH), :] = jnp.zeros((CH, NCOL), jnp.float32)
            x4buf[pl.ds(b * CH, CH), :] = jnp.zeros((CH, NCOL), jnp.float32)

    slot = jax.lax.rem(s, 2)
    nslot = jax.lax.rem(s + 1, 2)

    @pl.when(s + 1 < STEPS)
    def _():
        issue(s + 1, nslot, True)

    issue(s, slot, False)

    r = rs_ref[s]
    c = cs_ref[s]
    roff = pl.multiple_of(r * RH, 8)
    coff = pl.multiple_of(c * CH, 8)
    is_tail_col = c == lastc
    do_x4 = jnp.logical_or(s >= SWEEP, c <= r)

    for sl in (0, 1):
        @pl.when(jnp.logical_and(slot == sl, s < SWEEP))
        def _(sl=sl):
            x3buf[pl.ds(roff, RH), :] = x3buf[pl.ds(roff, RH), :] + jnp.dot(
                abuf[sl], y_ref[pl.ds(coff, CH), :],
                preferred_element_type=jnp.float32,
                precision=jax.lax.Precision.DEFAULT)

    @pl.when(jnp.logical_and(is_tail_col, s < SWEEP))
    def _():
        x3buf[pl.ds(roff, RH), :] = x3buf[pl.ds(roff, RH), :] + jnp.dot(
            tbuf[pl.ds(roff, RH), :], ytail_ref[...],
            preferred_element_type=jnp.float32)

    @pl.when(s == SWEEP - 1)
    def _():
        x3tail[...] = x3buf[pl.ds(TAILC, TAILW), :]
        x3buf[pl.ds(TAILC, NPAD - TAILC), :] = jnp.zeros(
            (NPAD - TAILC, NCOL), jnp.float32)

    for sl in (0, 1):
        @pl.when(jnp.logical_and(slot == sl, do_x4))
        def _(sl=sl):
            x4buf[pl.ds(roff, RH), :] = x4buf[pl.ds(roff, RH), :] + jnp.dot(
                abuf[sl], x3buf[pl.ds(coff, CH), :],
                preferred_element_type=jnp.float32,
                precision=jax.lax.Precision.DEFAULT)

    @pl.when(jnp.logical_and(is_tail_col, do_x4))
    def _():
        x4buf[pl.ds(roff, RH), :] = x4buf[pl.ds(roff, RH), :] + jnp.dot(
            tbuf[pl.ds(roff, RH), :], x3tail[...],
            preferred_element_type=jnp.float32)

    @pl.when(s == STEPS - 1)
    def _():
        for b in range(NB - 1):
            x_out_ref[pl.ds(b * CH, CH), :] = x4buf[pl.ds(b * CH, CH), :]
        x_out_ref[pl.ds((NB - 1) * CH, N - (NB - 1) * CH), :] = x4buf[
            pl.ds((NB - 1) * CH, N - (NB - 1) * CH), :]


def _decoder_kernel(xr_ref, xc_ref, out_ref):
    z = jax.lax.dot_general(xr_ref[...], xc_ref[...],
                            (((1,), (1,)), ((), ())),
                            preferred_element_type=jnp.float32)
    out_ref[...] = 0.5 * (jnp.tanh(0.5 * z) + 1.0)


def kernel(feat, adj, W1, W2):
    x = pl.pallas_call(
        _mega_kernel,
        grid_spec=pltpu.PrefetchScalarGridSpec(
            num_scalar_prefetch=2,
            grid=(STEPS,),
            in_specs=[
                pl.BlockSpec((N, 128), lambda s, rs, cs: (0, 0)),
                pl.BlockSpec((128, 128), lambda s, rs, cs: (0, 0)),
                pl.BlockSpec((128, NCOL), lambda s, rs, cs: (0, 0)),
                pl.BlockSpec(memory_space=pl.ANY),
            ],
            out_specs=pl.BlockSpec((N, NCOL), lambda s, rs, cs: (0, 0)),
            scratch_shapes=[
                pltpu.VMEM((2, RH, CH), jnp.float32),
                pltpu.VMEM((NPAD, TAILW), jnp.float32),
                pltpu.VMEM((NPAD, NCOL), jnp.float32),
                pltpu.VMEM((NPAD, NCOL), jnp.float32),
                pltpu.VMEM((TAILW, NCOL), jnp.float32),
                pltpu.VMEM((NPAD, NCOL), jnp.float32),
                pltpu.VMEM((TAILW, NCOL), jnp.float32),
                pltpu.SemaphoreType.DMA((2,)),
                pltpu.SemaphoreType.DMA,
            ],
        ),
        out_shape=jax.ShapeDtypeStruct((N, NCOL), jnp.float32),
        compiler_params=pltpu.CompilerParams(
            vmem_limit_bytes=63 * 1024 * 1024),
    )(jnp.asarray(_SCHED_R), jnp.asarray(_SCHED_C), feat, W1, W2, adj)

    a_rec = pl.pallas_call(
        _decoder_kernel,
        grid=(N // D_BLK,),
        in_specs=[
            pl.BlockSpec((D_BLK, NCOL), lambda i: (i, 0)),
            pl.BlockSpec((N, NCOL), lambda i: (0, 0)),
        ],
        out_specs=pl.BlockSpec((D_BLK, N), lambda i: (i, 0)),
        out_shape=jax.ShapeDtypeStruct((N, N), jnp.float32),
    )(x, x)

    return (x, a_rec)
